# Initial kernel scaffold; baseline (speedup 1.0000x reference)
import jax
import jax.numpy as jnp
from jax.experimental import pallas as pl


def kernel(text, lengths, table, W, b):
    raise NotImplementedError("write your pallas kernel here")


# speedup vs baseline: 1.8896x; 1.8896x over previous
import functools

import jax
import jax.numpy as jnp
from jax import lax
from jax.experimental import pallas as pl
from jax.experimental.pallas import tpu as pltpu
from jax.experimental.pallas import tpu_sc as plsc

VOCAB = 1000000
EMB = 32
OUT = 2
SEQ = 200
BATCH = 4096
LANES = 16

NUM_CORES = 2
NUM_SUBCORES = 16
NW = NUM_CORES * NUM_SUBCORES
BPW = BATCH // NW
H = SEQ // 2
INV_SEQ = 1.0 / SEQ


def _embed_pool_body(text_hbm, table_hbm, w_hbm, b_hbm, out_hbm,
                     idx_v, w_v, b_v, buf_v, pooled_t, out_t, sems):
    cid = lax.axis_index("c")
    sid = lax.axis_index("s")
    wid = sid * NUM_CORES + cid
    base = wid * BPW

    pltpu.sync_copy(text_hbm.at[pl.ds(base, BPW)], idx_v)
    pltpu.sync_copy(w_hbm, w_v)
    pltpu.sync_copy(b_hbm, b_v)

    lane = lax.iota(jnp.int32, LANES)
    bvec = b_v[pl.ds(0, LANES)]

    def fire(i, slot):
        pltpu.async_copy(table_hbm.at[idx_v.at[i, 0]],
                         buf_v.at[slot, pl.ds(0, H)], sems.at[slot])
        pltpu.async_copy(table_hbm.at[idx_v.at[i, 1]],
                         buf_v.at[slot, pl.ds(H, H)], sems.at[slot])

    def wait(i, slot):
        pltpu.make_async_copy(table_hbm.at[idx_v.at[i, 0]],
                              buf_v.at[slot, pl.ds(0, H)], sems.at[slot]).wait()
        pltpu.make_async_copy(table_hbm.at[idx_v.at[i, 1]],
                              buf_v.at[slot, pl.ds(H, H)], sems.at[slot]).wait()

    fire(0, 0)

    def elem(i, carry):
        slot = lax.rem(i, 2)

        @pl.when(i + 1 < BPW)
        def _():
            fire(i + 1, 1 - slot)

        wait(i, slot)

        def red(j, accs):
            a0, a1 = accs
            a0 = a0 + buf_v[slot, j, pl.ds(0, LANES)]
            a1 = a1 + buf_v[slot, j, pl.ds(LANES, LANES)]
            return a0, a1

        zeros = jnp.zeros((LANES,), jnp.float32)
        a0, a1 = lax.fori_loop(0, SEQ, red, (zeros, zeros), unroll=8)
        idx0 = lane * BPW + i
        plsc.store_scatter(pooled_t, [idx0], a0 * INV_SEQ)
        plsc.store_scatter(pooled_t, [idx0 + LANES * BPW], a1 * INV_SEQ)
        return carry

    lax.fori_loop(0, BPW, elem, 0)

    w_rows = [(w_v[o, pl.ds(0, LANES)], w_v[o, pl.ds(LANES, LANES)])
              for o in range(OUT)]

    def grp(g, carry):
        gbase = g * LANES
        for o in range(OUT):
            wa, wb = w_rows[o]
            acc = jnp.full((LANES,), bvec[o], jnp.float32)
            for d in range(LANES):
                acc = acc + pooled_t[pl.ds(d * BPW + gbase, LANES)] * wa[d]
                acc = acc + pooled_t[pl.ds((LANES + d) * BPW + gbase,
                                           LANES)] * wb[d]
            out_t[o, pl.ds(gbase, LANES)] = acc
        return carry

    lax.fori_loop(0, BPW // LANES, grp, 0)

    for o in range(OUT):
        pltpu.sync_copy(out_t.at[o], out_hbm.at[o, pl.ds(base, BPW)])


_embed_pool = functools.partial(
    pl.kernel,
    out_type=jax.ShapeDtypeStruct((OUT, BATCH), jnp.float32),
    mesh=plsc.VectorSubcoreMesh(core_axis_name="c", subcore_axis_name="s",
                                num_cores=NUM_CORES,
                                num_subcores=NUM_SUBCORES),
    scratch_types=[
        pltpu.VMEM((BPW, 2, H), jnp.int32),
        pltpu.VMEM((OUT, EMB), jnp.float32),
        pltpu.VMEM((LANES,), jnp.float32),
        pltpu.VMEM((2, SEQ, EMB), jnp.float32),
        pltpu.VMEM((EMB * BPW,), jnp.float32),
        pltpu.VMEM((OUT, BPW), jnp.float32),
        pltpu.SemaphoreType.DMA((2,)),
    ],
    compiler_params=pltpu.CompilerParams(needs_layout_passes=False,
                                         use_tc_tiling_on_sc=False),
)(_embed_pool_body)


def kernel(text, lengths, table, W, b):
    del lengths
    text_r = text.T.reshape(BATCH, 2, H)
    bpad = jnp.zeros((LANES,), jnp.float32).at[:OUT].set(b)
    return _embed_pool(text_r, table, W, bpad).T


# speedup vs baseline: 2.0250x; 1.0717x over previous
import functools

import jax
import jax.numpy as jnp
from jax import lax
from jax.experimental import pallas as pl
from jax.experimental.pallas import tpu as pltpu
from jax.experimental.pallas import tpu_sc as plsc

VOCAB = 1000000
EMB = 32
OUT = 2
SEQ = 200
BATCH = 4096
LANES = 16

NUM_CORES = 2
NUM_SUBCORES = 16
NW = NUM_CORES * NUM_SUBCORES
BPW = BATCH // NW
H = SEQ // 2
G = 4
NSLOT = 3
NCHUNK = BPW // G
INV_SEQ = 1.0 / SEQ


def _embed_pool_body(text_hbm, table_hbm, w_hbm, b_hbm, out_hbm,
                     idx_v, w_v, b_v, buf_v, pooled_t, out_t, sems):
    cid = lax.axis_index("c")
    sid = lax.axis_index("s")
    wid = sid * NUM_CORES + cid
    base = wid * BPW

    pltpu.sync_copy(text_hbm.at[pl.ds(base, BPW)], idx_v)
    pltpu.sync_copy(w_hbm, w_v)
    pltpu.sync_copy(b_hbm, b_v)

    lane = lax.iota(jnp.int32, LANES)
    bvec = b_v[pl.ds(0, LANES)]

    def fire(c, slot):
        for e in range(G):
            i = c * G + e
            pltpu.async_copy(table_hbm.at[idx_v.at[i, 0]],
                             buf_v.at[slot, pl.ds(e * SEQ, H)],
                             sems.at[slot])
            pltpu.async_copy(table_hbm.at[idx_v.at[i, 1]],
                             buf_v.at[slot, pl.ds(e * SEQ + H, H)],
                             sems.at[slot])

    def wait(c, slot):
        for e in range(G):
            i = c * G + e
            pltpu.make_async_copy(table_hbm.at[idx_v.at[i, 0]],
                                  buf_v.at[slot, pl.ds(e * SEQ, H)],
                                  sems.at[slot]).wait()
            pltpu.make_async_copy(table_hbm.at[idx_v.at[i, 1]],
                                  buf_v.at[slot, pl.ds(e * SEQ + H, H)],
                                  sems.at[slot]).wait()

    fire(0, 0)
    fire(1, 1)

    def chunk(c, carry):
        slot = lax.rem(c, NSLOT)

        @pl.when(c + 2 < NCHUNK)
        def _():
            fire(c + 2, lax.rem(c + 2, NSLOT))

        wait(c, slot)

        zeros = jnp.zeros((LANES,), jnp.float32)
        for e in range(G):
            def red(j, accs, e=e):
                a0, a1 = accs
                a0 = a0 + buf_v[slot, e * SEQ + j, pl.ds(0, LANES)]
                a1 = a1 + buf_v[slot, e * SEQ + j, pl.ds(LANES, LANES)]
                return a0, a1

            a0, a1 = lax.fori_loop(0, SEQ, red, (zeros, zeros), unroll=8)
            idx0 = lane * BPW + c * G + e
            plsc.store_scatter(pooled_t, [idx0], a0 * INV_SEQ)
            plsc.store_scatter(pooled_t, [idx0 + LANES * BPW], a1 * INV_SEQ)
        return carry

    lax.fori_loop(0, NCHUNK, chunk, 0)

    w_rows = [(w_v[o, pl.ds(0, LANES)], w_v[o, pl.ds(LANES, LANES)])
              for o in range(OUT)]

    def grp(g, carry):
        gbase = g * LANES
        for o in range(OUT):
            wa, wb = w_rows[o]
            acc = jnp.full((LANES,), bvec[o], jnp.float32)
            for d in range(LANES):
                acc = acc + pooled_t[pl.ds(d * BPW + gbase, LANES)] * wa[d]
                acc = acc + pooled_t[pl.ds((LANES + d) * BPW + gbase,
                                           LANES)] * wb[d]
            out_t[o, pl.ds(gbase, LANES)] = acc
        return carry

    lax.fori_loop(0, BPW // LANES, grp, 0)

    for o in range(OUT):
        pltpu.sync_copy(out_t.at[o], out_hbm.at[o, pl.ds(base, BPW)])


_embed_pool = functools.partial(
    pl.kernel,
    out_type=jax.ShapeDtypeStruct((OUT, BATCH), jnp.float32),
    mesh=plsc.VectorSubcoreMesh(core_axis_name="c", subcore_axis_name="s",
                                num_cores=NUM_CORES,
                                num_subcores=NUM_SUBCORES),
    scratch_types=[
        pltpu.VMEM((BPW, 2, H), jnp.int32),
        pltpu.VMEM((OUT, EMB), jnp.float32),
        pltpu.VMEM((LANES,), jnp.float32),
        pltpu.VMEM((NSLOT, G * SEQ, EMB), jnp.float32),
        pltpu.VMEM((EMB * BPW,), jnp.float32),
        pltpu.VMEM((OUT, BPW), jnp.float32),
        pltpu.SemaphoreType.DMA((NSLOT,)),
    ],
    compiler_params=pltpu.CompilerParams(needs_layout_passes=False,
                                         use_tc_tiling_on_sc=False),
)(_embed_pool_body)


def kernel(text, lengths, table, W, b):
    del lengths
    text_r = text.T.reshape(BATCH, 2, H)
    bpad = jnp.zeros((LANES,), jnp.float32).at[:OUT].set(b)
    return _embed_pool(text_r, table, W, bpad).T
